# baseline (speedup 1.0000x reference)
import jax
import jax.numpy as jnp
from jax import lax
from jax.experimental import pallas as pl
from jax.experimental.pallas import tpu as pltpu

_WEIGHT1 = 1.0
_WEIGHT2 = 0.5
_SIGMA1 = 1.0
_SIGMA2 = 2.0
_WEIGHT_MARGIN = 0.8

_BR = 2048


def _body(preds_ref, tgt_ref, mar_ref, sco_ref, out_ref, acc_ref, sc_ref):
    i = pl.program_id(0)
    n_steps = pl.num_programs(0)

    @pl.when(i == 0)
    def _():
        acc_ref[...] = jnp.zeros_like(acc_ref)
        sc_ref[...] = jnp.zeros_like(sc_ref)

    sl = pl.ds(i * _BR, _BR)
    t = tgt_ref[sl]
    m = mar_ref[sl]
    s = sco_ref[sl]

    m2 = m * m
    w = (jnp.where(m > 0, _WEIGHT1 * jnp.exp(-_SIGMA1 * m2), 0.0)
         + jnp.where(m < 0, _WEIGHT2 * jnp.exp(-_SIGMA2 * m2), 0.0))

    cols = lax.broadcasted_iota(jnp.int32, preds_ref.shape, 1)
    pf = preds_ref[...].astype(jnp.float32)
    contrib = jnp.where(cols == t[:, None], w[:, None] * pf, 0.0)
    acc_ref[...] += jnp.sum(
        contrib.reshape(_BR // 8, 8, preds_ref.shape[1]), axis=0)

    neg = s < 0
    s_neg = jnp.sum(jnp.where(neg, s, 0.0))
    c_neg = jnp.sum(jnp.where(neg, 1.0, 0.0))
    r = lax.broadcasted_iota(jnp.int32, sc_ref.shape, 0)
    c = lax.broadcasted_iota(jnp.int32, sc_ref.shape, 1)
    first = c == 0
    sc_ref[...] += (jnp.where((r == 1) & first, s_neg, 0.0)
                    + jnp.where((r == 2) & first, c_neg, 0.0))

    @pl.when(i == n_steps - 1)
    def _():
        w_loss = jnp.sum(acc_ref[...])
        neg_sum = jnp.sum(jnp.where((r == 1) & first, sc_ref[...], 0.0))
        neg_cnt = jnp.sum(jnp.where((r == 2) & first, sc_ref[...], 0.0))
        b_rows = _BR * n_steps
        out_ref[0] = (-w_loss / b_rows
                      + _WEIGHT_MARGIN * (neg_sum / neg_cnt))


def kernel(preds, targets, margin, score):
    B, C = preds.shape
    grid = B // _BR
    preds = preds.astype(jnp.bfloat16)
    out = pl.pallas_call(
        _body,
        grid=(grid,),
        in_specs=[
            pl.BlockSpec((_BR, C), lambda i: (i, 0)),
            pl.BlockSpec((B,), lambda i: (0,)),
            pl.BlockSpec((B,), lambda i: (0,)),
            pl.BlockSpec((B,), lambda i: (0,)),
        ],
        out_specs=pl.BlockSpec(memory_space=pltpu.SMEM),
        out_shape=jax.ShapeDtypeStruct((1,), jnp.float32),
        scratch_shapes=[
            pltpu.VMEM((8, C), jnp.float32),
            pltpu.VMEM((8, 128), jnp.float32),
        ],
        compiler_params=pltpu.CompilerParams(
            dimension_semantics=("arbitrary",),
        ),
    )(preds, targets, margin, score)
    return out[0]


# speedup vs baseline: 1.1368x; 1.1368x over previous
import jax
import jax.numpy as jnp
from jax import lax
from jax.experimental import pallas as pl
from jax.experimental.pallas import tpu as pltpu

_WEIGHT1 = 1.0
_WEIGHT2 = 0.5
_SIGMA1 = 1.0
_SIGMA2 = 2.0
_WEIGHT_MARGIN = 0.8

_BR = 2048


def _body(preds_ref, tgt_ref, mar_ref, sco_ref, out_ref, acc_ref, sc_ref):
    i = pl.program_id(0)
    n_steps = pl.num_programs(0)

    @pl.when(i == 0)
    def _():
        acc_ref[...] = jnp.zeros_like(acc_ref)
        sc_ref[...] = jnp.zeros_like(sc_ref)

    sl = pl.ds(i * _BR, _BR)
    t = tgt_ref[sl]
    m = mar_ref[sl]
    s = sco_ref[sl]

    m2 = m * m
    w = (jnp.where(m > 0, _WEIGHT1 * jnp.exp(-_SIGMA1 * m2), 0.0)
         + jnp.where(m < 0, _WEIGHT2 * jnp.exp(-_SIGMA2 * m2), 0.0))

    cols = lax.broadcasted_iota(jnp.int32, preds_ref.shape, 1)
    contrib = jnp.where(cols == t[:, None], w[:, None] * preds_ref[...], 0.0)
    acc_ref[...] += jnp.sum(
        contrib.reshape(_BR // 8, 8, preds_ref.shape[1]), axis=0)

    neg = s < 0
    s_neg = jnp.sum(jnp.where(neg, s, 0.0))
    c_neg = jnp.sum(jnp.where(neg, 1.0, 0.0))
    r = lax.broadcasted_iota(jnp.int32, sc_ref.shape, 0)
    c = lax.broadcasted_iota(jnp.int32, sc_ref.shape, 1)
    first = c == 0
    sc_ref[...] += (jnp.where((r == 1) & first, s_neg, 0.0)
                    + jnp.where((r == 2) & first, c_neg, 0.0))

    @pl.when(i == n_steps - 1)
    def _():
        w_loss = jnp.sum(acc_ref[...])
        neg_sum = jnp.sum(jnp.where((r == 1) & first, sc_ref[...], 0.0))
        neg_cnt = jnp.sum(jnp.where((r == 2) & first, sc_ref[...], 0.0))
        b_rows = _BR * n_steps
        out_ref[0] = (-w_loss / b_rows
                      + _WEIGHT_MARGIN * (neg_sum / neg_cnt))


def kernel(preds, targets, margin, score):
    B, C = preds.shape
    grid = B // _BR
    out = pl.pallas_call(
        _body,
        grid=(grid,),
        in_specs=[
            pl.BlockSpec((_BR, C), lambda i: (i, 0)),
            pl.BlockSpec((B,), lambda i: (0,)),
            pl.BlockSpec((B,), lambda i: (0,)),
            pl.BlockSpec((B,), lambda i: (0,)),
        ],
        out_specs=pl.BlockSpec(memory_space=pltpu.SMEM),
        out_shape=jax.ShapeDtypeStruct((1,), jnp.float32),
        scratch_shapes=[
            pltpu.VMEM((8, C), jnp.float32),
            pltpu.VMEM((8, 128), jnp.float32),
        ],
        compiler_params=pltpu.CompilerParams(
            dimension_semantics=("arbitrary",),
        ),
    )(preds, targets, margin, score)
    return out[0]
